# baseline (speedup 1.0000x reference)
import functools

import jax
import jax.numpy as jnp
from jax import lax
from jax.experimental import pallas as pl
from jax.experimental.pallas import tpu as pltpu
from jax.experimental.pallas import tpu_sc as plsc

N = 20000
S = 2048
SPACE = 3
FUN = 64
CH = 256
K = 16

L = 16
NC = 2
NS = 16
NW = NC * NS
SN_PER_W = S // NW
CHUNKS = N // L
GROUPS = SN_PER_W // L
FV = FUN // L
FPAD = 128
DPAD = 128


def _splat_lane(vec, lane_iota, lane):
    m = lane_iota == lane
    val = jnp.sum(jnp.where(m, vec, jnp.float32(0.0)))
    return jnp.full((L,), val, jnp.float32)


def _round_bf16(v):
    u = lax.bitcast_convert_type(v, jnp.uint32)
    lsb = lax.shift_right_logical(u, jnp.uint32(16)) & jnp.uint32(1)
    r = u + jnp.uint32(0x7FFF) + lsb
    return lax.bitcast_convert_type(r & jnp.uint32(0xFFFF0000), jnp.float32)


def _sc_body(x_hbm, y_hbm, z_hbm, sidx_hbm, fun_hbm, feats_hbm,
             x_v, y_v, z_v, pn_v, sidx_v, sp_v, sp2_v, feats_v,
             nidx_v, rows_v, sem):
    cid = lax.axis_index("c")
    sid = lax.axis_index("s")
    wid = sid * NC + cid
    base = wid * SN_PER_W

    pltpu.sync_copy(x_hbm, x_v)
    pltpu.sync_copy(y_hbm, y_v)
    pltpu.sync_copy(z_hbm, z_v)
    pltpu.sync_copy(sidx_hbm.at[pl.ds(base, SN_PER_W)], sidx_v)

    for g in range(GROUPS):
        idx16 = sidx_v[pl.ds(g * L, L)]
        sx = plsc.load_gather(x_v, [idx16])
        sy = plsc.load_gather(y_v, [idx16])
        sz = plsc.load_gather(z_v, [idx16])
        sp_v[0, pl.ds(g * L, L)] = sx
        sp_v[1, pl.ds(g * L, L)] = sy
        sp_v[2, pl.ds(g * L, L)] = sz
        sxr = _round_bf16(sx)
        syr = _round_bf16(sy)
        szr = _round_bf16(sz)
        sp2_v[0, pl.ds(g * L, L)] = sxr + sxr
        sp2_v[1, pl.ds(g * L, L)] = syr + syr
        sp2_v[2, pl.ds(g * L, L)] = szr + szr

    def pn_step(c, carry):
        off = c * L
        xv = x_v[pl.ds(off, L)]
        yv = y_v[pl.ds(off, L)]
        zv = z_v[pl.ds(off, L)]
        pn_v[pl.ds(off, L)] = xv * xv + yv * yv + zv * zv
        x_v[pl.ds(off, L)] = _round_bf16(xv)
        y_v[pl.ds(off, L)] = _round_bf16(yv)
        z_v[pl.ds(off, L)] = _round_bf16(zv)
        return carry
    lax.fori_loop(0, CHUNKS, pn_step, 0)

    lane_iota = lax.iota(jnp.int32, L)
    inf16 = jnp.full((L,), jnp.inf, jnp.float32)
    zero16 = jnp.zeros((L,), jnp.float32)

    for g in range(GROUPS):
        sxv = sp_v[0, pl.ds(g * L, L)]
        syv = sp_v[1, pl.ds(g * L, L)]
        szv = sp_v[2, pl.ds(g * L, L)]
        sx2v = sp2_v[0, pl.ds(g * L, L)]
        sy2v = sp2_v[1, pl.ds(g * L, L)]
        sz2v = sp2_v[2, pl.ds(g * L, L)]

        def sn_step(lane, carry, sxv=sxv, syv=syv, szv=szv,
                    sx2v=sx2v, sy2v=sy2v, sz2v=sz2v, g=g):
            sx2 = _splat_lane(sx2v, lane_iota, lane)
            sy2 = _splat_lane(sy2v, lane_iota, lane)
            sz2 = _splat_lane(sz2v, lane_iota, lane)

            def chunk_step(c, st):
                bd, bi, thr = st
                off = c * L
                xv = x_v[pl.ds(off, L)]
                yv = y_v[pl.ds(off, L)]
                zv = z_v[pl.ds(off, L)]
                pnv = pn_v[pl.ds(off, L)]
                t = sx2 * xv + sy2 * yv + sz2 * zv
                d = pnv - t
                hit = jnp.any(d < thr)

                def do_merge(args):
                    bd, bi, _ = args
                    idxv = lane_iota + off
                    nd, ni = plsc.sort_key_val(d, idxv)
                    ndr = lax.rev(nd, (0,))
                    nir = lax.rev(ni, (0,))
                    take = bd <= ndr
                    lo_d = jnp.where(take, bd, ndr)
                    lo_i = jnp.where(take, bi, nir)
                    bd2, bi2 = plsc.sort_key_val(lo_d, lo_i)
                    thr2 = jnp.full((L,), jnp.max(bd2), jnp.float32)
                    return bd2, bi2, thr2

                return lax.cond(hit, do_merge, lambda a: a, (bd, bi, thr))

            bd, bi, _ = lax.fori_loop(
                0, CHUNKS, chunk_step,
                (inf16, jnp.zeros((L,), jnp.int32), inf16))

            nidx_v[...] = bi
            pltpu.async_copy(fun_hbm.at[nidx_v], rows_v, sem).wait()
            acc = [jnp.zeros((L,), jnp.float32) for _ in range(FV)]
            for r in range(K):
                for j in range(FV):
                    acc[j] = acc[j] + rows_v[r, pl.ds(j * L, L)]
            sn = g * L + lane
            scale = jnp.float32(1.0 / K)
            for j in range(FV):
                feats_v[sn, pl.ds(j * L, L)] = acc[j] * scale
            posv = jnp.where(
                lane_iota == 0, _splat_lane(sxv, lane_iota, lane),
                jnp.where(lane_iota == 1, _splat_lane(syv, lane_iota, lane),
                          jnp.where(lane_iota == 2,
                                    _splat_lane(szv, lane_iota, lane),
                                    zero16)))
            feats_v[sn, pl.ds(FUN, L)] = posv
            for j in range(FV + 1, DPAD // L):
                feats_v[sn, pl.ds(j * L, L)] = zero16
            return carry

        lax.fori_loop(0, L, sn_step, 0)

    pltpu.sync_copy(feats_v, feats_hbm.at[pl.ds(base, SN_PER_W)])


def _mm_body(f_ref, w_ref, b_ref, o_ref):
    o_ref[...] = (
        jax.lax.dot_general(
            f_ref[...], w_ref[...], (((1,), (0,)), ((), ())),
            preferred_element_type=jnp.float32,
            precision=jax.lax.Precision.HIGHEST)
        + b_ref[...]
    )


def kernel(pos, fun, supernode_idx, W, b):
    pos32 = pos.astype(jnp.float32)
    xs, ys, zs = pos32[:, 0], pos32[:, 1], pos32[:, 2]
    sidx = supernode_idx.astype(jnp.int32)
    fun32 = jnp.pad(fun.astype(jnp.float32), ((0, 0), (0, FPAD - FUN)))

    mesh = plsc.VectorSubcoreMesh(
        core_axis_name="c", subcore_axis_name="s",
        num_cores=NC, num_subcores=NS)
    sc = pl.kernel(
        _sc_body,
        out_type=[
            jax.ShapeDtypeStruct((S, DPAD), jnp.float32),
        ],
        mesh=mesh,
        scratch_types=[
            pltpu.VMEM((N,), jnp.float32),
            pltpu.VMEM((N,), jnp.float32),
            pltpu.VMEM((N,), jnp.float32),
            pltpu.VMEM((N,), jnp.float32),
            pltpu.VMEM((SN_PER_W,), jnp.int32),
            pltpu.VMEM((SPACE, SN_PER_W), jnp.float32),
            pltpu.VMEM((SPACE, SN_PER_W), jnp.float32),
            pltpu.VMEM((SN_PER_W, DPAD), jnp.float32),
            pltpu.VMEM((K,), jnp.int32),
            pltpu.VMEM((K, FPAD), jnp.float32),
            pltpu.SemaphoreType.DMA,
        ],
        compiler_params=pltpu.CompilerParams(needs_layout_passes=False),
    )
    (feats,) = sc(xs, ys, zs, sidx, fun32)

    W32 = W.astype(jnp.float32)
    Wp = jnp.concatenate(
        [W32[:, SPACE:], W32[:, :SPACE],
         jnp.zeros((CH, DPAD - SPACE - FUN), jnp.float32)], axis=1).T

    latent = pl.pallas_call(
        _mm_body,
        out_shape=jax.ShapeDtypeStruct((S, CH), jnp.float32),
    )(feats, Wp, b.astype(jnp.float32).reshape(1, CH))
    return latent


# speedup vs baseline: 2.1900x; 2.1900x over previous
import functools

import jax
import jax.numpy as jnp
from jax import lax
from jax.experimental import pallas as pl
from jax.experimental.pallas import tpu as pltpu
from jax.experimental.pallas import tpu_sc as plsc

N = 20000
S = 2048
SPACE = 3
FUN = 64
CH = 256
K = 16

L = 16
NC = 2
NS = 16
NW = NC * NS
SN_PER_W = S // NW
CHUNKS = N // L
GCH = 10
GROUPS = SN_PER_W // L
FV = FUN // L
FPAD = 128
DPAD = 128


def _splat_lane(vec, lane_iota, lane):
    m = lane_iota == lane
    val = jnp.sum(jnp.where(m, vec, jnp.float32(0.0)))
    return jnp.full((L,), val, jnp.float32)


def _round_bf16(v):
    u = lax.bitcast_convert_type(v, jnp.uint32)
    lsb = lax.shift_right_logical(u, jnp.uint32(16)) & jnp.uint32(1)
    r = u + jnp.uint32(0x7FFF) + lsb
    return lax.bitcast_convert_type(r & jnp.uint32(0xFFFF0000), jnp.float32)


def _sc_body(x_hbm, y_hbm, z_hbm, sidx_hbm, fun_hbm, feats_hbm,
             x_v, y_v, z_v, pn_v, sidx_v, sp_v, sp2_v, feats_v,
             nidx_v, rows_v, sem):
    cid = lax.axis_index("c")
    sid = lax.axis_index("s")
    wid = sid * NC + cid
    base = wid * SN_PER_W

    pltpu.sync_copy(x_hbm, x_v)
    pltpu.sync_copy(y_hbm, y_v)
    pltpu.sync_copy(z_hbm, z_v)
    pltpu.sync_copy(sidx_hbm.at[pl.ds(base, SN_PER_W)], sidx_v)

    for g in range(GROUPS):
        idx16 = sidx_v[pl.ds(g * L, L)]
        sx = plsc.load_gather(x_v, [idx16])
        sy = plsc.load_gather(y_v, [idx16])
        sz = plsc.load_gather(z_v, [idx16])
        sp_v[0, pl.ds(g * L, L)] = sx
        sp_v[1, pl.ds(g * L, L)] = sy
        sp_v[2, pl.ds(g * L, L)] = sz
        sxr = _round_bf16(sx)
        syr = _round_bf16(sy)
        szr = _round_bf16(sz)
        sp2_v[0, pl.ds(g * L, L)] = sxr + sxr
        sp2_v[1, pl.ds(g * L, L)] = syr + syr
        sp2_v[2, pl.ds(g * L, L)] = szr + szr

    def pn_step(c, carry):
        off = c * L
        xv = x_v[pl.ds(off, L)]
        yv = y_v[pl.ds(off, L)]
        zv = z_v[pl.ds(off, L)]
        pn_v[pl.ds(off, L)] = xv * xv + yv * yv + zv * zv
        x_v[pl.ds(off, L)] = _round_bf16(xv)
        y_v[pl.ds(off, L)] = _round_bf16(yv)
        z_v[pl.ds(off, L)] = _round_bf16(zv)
        return carry
    lax.fori_loop(0, CHUNKS, pn_step, 0)

    lane_iota = lax.iota(jnp.int32, L)
    inf16 = jnp.full((L,), jnp.inf, jnp.float32)
    zero16 = jnp.zeros((L,), jnp.float32)

    for g in range(GROUPS):
        sxv = sp_v[0, pl.ds(g * L, L)]
        syv = sp_v[1, pl.ds(g * L, L)]
        szv = sp_v[2, pl.ds(g * L, L)]
        sx2v = sp2_v[0, pl.ds(g * L, L)]
        sy2v = sp2_v[1, pl.ds(g * L, L)]
        sz2v = sp2_v[2, pl.ds(g * L, L)]

        def sn_step(lane, carry, sxv=sxv, syv=syv, szv=szv,
                    sx2v=sx2v, sy2v=sy2v, sz2v=sz2v, g=g):
            sx2 = _splat_lane(sx2v, lane_iota, lane)
            sy2 = _splat_lane(sy2v, lane_iota, lane)
            sz2 = _splat_lane(sz2v, lane_iota, lane)

            def group_step(gi, st):
                bd, bi, thr = st
                off0 = gi * (GCH * L)
                ds = []
                dmin = None
                for k in range(GCH):
                    off = off0 + k * L
                    xv = x_v[pl.ds(off, L)]
                    yv = y_v[pl.ds(off, L)]
                    zv = z_v[pl.ds(off, L)]
                    pnv = pn_v[pl.ds(off, L)]
                    t = sx2 * xv + sy2 * yv + sz2 * zv
                    d = pnv - t
                    ds.append(d)
                    dmin = d if dmin is None else jnp.minimum(dmin, d)
                ghit = jnp.any(dmin < thr)

                def do_group(args):
                    bd, bi, thr = args
                    for k in range(GCH):
                        d = ds[k]
                        hit = jnp.any(d < thr)

                        def do_merge(args, d=d, k=k):
                            bd, bi, _ = args
                            idxv = lane_iota + (off0 + k * L)
                            nd, ni = plsc.sort_key_val(d, idxv)
                            ndr = lax.rev(nd, (0,))
                            nir = lax.rev(ni, (0,))
                            take = bd <= ndr
                            lo_d = jnp.where(take, bd, ndr)
                            lo_i = jnp.where(take, bi, nir)
                            bd2, bi2 = plsc.sort_key_val(lo_d, lo_i)
                            thr2 = jnp.full((L,), jnp.max(bd2), jnp.float32)
                            return bd2, bi2, thr2

                        bd, bi, thr = lax.cond(
                            hit, do_merge, lambda a: a, (bd, bi, thr))
                    return bd, bi, thr

                return lax.cond(ghit, do_group, lambda a: a, (bd, bi, thr))

            bd, bi, _ = lax.fori_loop(
                0, CHUNKS // GCH, group_step,
                (inf16, jnp.zeros((L,), jnp.int32), inf16))

            nidx_v[...] = bi
            pltpu.async_copy(fun_hbm.at[nidx_v], rows_v, sem).wait()
            acc = [jnp.zeros((L,), jnp.float32) for _ in range(FV)]
            for r in range(K):
                for j in range(FV):
                    acc[j] = acc[j] + rows_v[r, pl.ds(j * L, L)]
            sn = g * L + lane
            scale = jnp.float32(1.0 / K)
            for j in range(FV):
                feats_v[sn, pl.ds(j * L, L)] = acc[j] * scale
            posv = jnp.where(
                lane_iota == 0, _splat_lane(sxv, lane_iota, lane),
                jnp.where(lane_iota == 1, _splat_lane(syv, lane_iota, lane),
                          jnp.where(lane_iota == 2,
                                    _splat_lane(szv, lane_iota, lane),
                                    zero16)))
            feats_v[sn, pl.ds(FUN, L)] = posv
            for j in range(FV + 1, DPAD // L):
                feats_v[sn, pl.ds(j * L, L)] = zero16
            return carry

        lax.fori_loop(0, L, sn_step, 0)

    pltpu.sync_copy(feats_v, feats_hbm.at[pl.ds(base, SN_PER_W)])


def _mm_body(f_ref, w_ref, b_ref, o_ref):
    o_ref[...] = (
        jax.lax.dot_general(
            f_ref[...], w_ref[...], (((1,), (0,)), ((), ())),
            preferred_element_type=jnp.float32,
            precision=jax.lax.Precision.HIGHEST)
        + b_ref[...]
    )


def kernel(pos, fun, supernode_idx, W, b):
    pos32 = pos.astype(jnp.float32)
    xs, ys, zs = pos32[:, 0], pos32[:, 1], pos32[:, 2]
    sidx = supernode_idx.astype(jnp.int32)
    fun32 = jnp.pad(fun.astype(jnp.float32), ((0, 0), (0, FPAD - FUN)))

    mesh = plsc.VectorSubcoreMesh(
        core_axis_name="c", subcore_axis_name="s",
        num_cores=NC, num_subcores=NS)
    sc = pl.kernel(
        _sc_body,
        out_type=[
            jax.ShapeDtypeStruct((S, DPAD), jnp.float32),
        ],
        mesh=mesh,
        scratch_types=[
            pltpu.VMEM((N,), jnp.float32),
            pltpu.VMEM((N,), jnp.float32),
            pltpu.VMEM((N,), jnp.float32),
            pltpu.VMEM((N,), jnp.float32),
            pltpu.VMEM((SN_PER_W,), jnp.int32),
            pltpu.VMEM((SPACE, SN_PER_W), jnp.float32),
            pltpu.VMEM((SPACE, SN_PER_W), jnp.float32),
            pltpu.VMEM((SN_PER_W, DPAD), jnp.float32),
            pltpu.VMEM((K,), jnp.int32),
            pltpu.VMEM((K, FPAD), jnp.float32),
            pltpu.SemaphoreType.DMA,
        ],
        compiler_params=pltpu.CompilerParams(needs_layout_passes=False),
    )
    (feats,) = sc(xs, ys, zs, sidx, fun32)

    W32 = W.astype(jnp.float32)
    Wp = jnp.concatenate(
        [W32[:, SPACE:], W32[:, :SPACE],
         jnp.zeros((CH, DPAD - SPACE - FUN), jnp.float32)], axis=1).T

    latent = pl.pallas_call(
        _mm_body,
        out_shape=jax.ShapeDtypeStruct((S, CH), jnp.float32),
    )(feats, Wp, b.astype(jnp.float32).reshape(1, CH))
    return latent


# speedup vs baseline: 3.0629x; 1.3986x over previous
import functools

import jax
import jax.numpy as jnp
from jax import lax
from jax.experimental import pallas as pl
from jax.experimental.pallas import tpu as pltpu
from jax.experimental.pallas import tpu_sc as plsc

N = 20000
S = 2048
SPACE = 3
FUN = 64
CH = 256
K = 16

L = 16
NC = 2
NS = 16
NW = NC * NS
SN_PER_W = S // NW
CHUNKS = N // L
WARM = 50
GCH = 25
NSUB = 5
SUBCH = GCH // NSUB
NGROUPS = (CHUNKS - WARM) // GCH
GROUPS = SN_PER_W // L
FV = FUN // L
FPAD = 128
DPAD = 128


def _splat_lane(vec, lane_iota, lane):
    m = lane_iota == lane
    val = jnp.sum(jnp.where(m, vec, jnp.float32(0.0)))
    return jnp.full((L,), val, jnp.float32)


def _round_bf16(v):
    u = lax.bitcast_convert_type(v, jnp.uint32)
    lsb = lax.shift_right_logical(u, jnp.uint32(16)) & jnp.uint32(1)
    r = u + jnp.uint32(0x7FFF) + lsb
    return lax.bitcast_convert_type(r & jnp.uint32(0xFFFF0000), jnp.float32)


def _sc_body(x_hbm, y_hbm, z_hbm, sidx_hbm, fun_hbm, feats_hbm,
             x_v, y_v, z_v, pn_v, sidx_v, sp_v, sp2_v, feats_v,
             nidx_v, rows_v, sem):
    cid = lax.axis_index("c")
    sid = lax.axis_index("s")
    wid = sid * NC + cid
    base = wid * SN_PER_W

    pltpu.sync_copy(x_hbm, x_v)
    pltpu.sync_copy(y_hbm, y_v)
    pltpu.sync_copy(z_hbm, z_v)
    pltpu.sync_copy(sidx_hbm.at[pl.ds(base, SN_PER_W)], sidx_v)

    for g in range(GROUPS):
        idx16 = sidx_v[pl.ds(g * L, L)]
        sx = plsc.load_gather(x_v, [idx16])
        sy = plsc.load_gather(y_v, [idx16])
        sz = plsc.load_gather(z_v, [idx16])
        sp_v[0, pl.ds(g * L, L)] = sx
        sp_v[1, pl.ds(g * L, L)] = sy
        sp_v[2, pl.ds(g * L, L)] = sz
        sxr = _round_bf16(sx)
        syr = _round_bf16(sy)
        szr = _round_bf16(sz)
        sp2_v[0, pl.ds(g * L, L)] = sxr + sxr
        sp2_v[1, pl.ds(g * L, L)] = syr + syr
        sp2_v[2, pl.ds(g * L, L)] = szr + szr

    def pn_step(c, carry):
        off = c * L
        xv = x_v[pl.ds(off, L)]
        yv = y_v[pl.ds(off, L)]
        zv = z_v[pl.ds(off, L)]
        pn_v[pl.ds(off, L)] = xv * xv + yv * yv + zv * zv
        x_v[pl.ds(off, L)] = _round_bf16(xv)
        y_v[pl.ds(off, L)] = _round_bf16(yv)
        z_v[pl.ds(off, L)] = _round_bf16(zv)
        return carry
    lax.fori_loop(0, CHUNKS, pn_step, 0)

    lane_iota = lax.iota(jnp.int32, L)
    inf16 = jnp.full((L,), jnp.inf, jnp.float32)
    zero16 = jnp.zeros((L,), jnp.float32)

    for g in range(GROUPS):
        sxv = sp_v[0, pl.ds(g * L, L)]
        syv = sp_v[1, pl.ds(g * L, L)]
        szv = sp_v[2, pl.ds(g * L, L)]
        sx2v = sp2_v[0, pl.ds(g * L, L)]
        sy2v = sp2_v[1, pl.ds(g * L, L)]
        sz2v = sp2_v[2, pl.ds(g * L, L)]

        def sn_step(lane, carry, sxv=sxv, syv=syv, szv=szv,
                    sx2v=sx2v, sy2v=sy2v, sz2v=sz2v, g=g):
            sx2 = _splat_lane(sx2v, lane_iota, lane)
            sy2 = _splat_lane(sy2v, lane_iota, lane)
            sz2 = _splat_lane(sz2v, lane_iota, lane)

            def dist(off):
                xv = x_v[pl.ds(off, L)]
                yv = y_v[pl.ds(off, L)]
                zv = z_v[pl.ds(off, L)]
                pnv = pn_v[pl.ds(off, L)]
                t = sx2 * xv + sy2 * yv + sz2 * zv
                return pnv - t

            def merge(st, d, off):
                bd, bi, _ = st
                idxv = lane_iota + off
                nd, ni = plsc.sort_key_val(d, idxv)
                ndr = lax.rev(nd, (0,))
                nir = lax.rev(ni, (0,))
                take = bd <= ndr
                lo_d = jnp.where(take, bd, ndr)
                lo_i = jnp.where(take, bi, nir)
                bd2, bi2 = plsc.sort_key_val(lo_d, lo_i)
                thr2 = plsc.cummax(lax.rev(plsc.cummax(lo_d), (0,)))
                return bd2, bi2, thr2

            def warm_step(c, st):
                off = c * L
                d = dist(off)
                hit = jnp.any(d < st[2])
                return lax.cond(
                    hit, lambda a: merge(a, d, off), lambda a: a, st)

            st = lax.fori_loop(
                0, WARM, warm_step,
                (inf16, jnp.zeros((L,), jnp.int32), inf16))

            def group_step(gi, st):
                off0 = (WARM + gi * GCH) * L
                sub_data = []
                gmin = None
                for s in range(NSUB):
                    dmin_s = None
                    ds_s = []
                    for k in range(SUBCH):
                        off = off0 + (s * SUBCH + k) * L
                        d = dist(off)
                        ds_s.append((d, off))
                        dmin_s = d if dmin_s is None else jnp.minimum(dmin_s, d)
                    sub_data.append((dmin_s, ds_s))
                    gmin = dmin_s if gmin is None else jnp.minimum(gmin, dmin_s)
                ghit = jnp.any(gmin < st[2])

                def do_group(st):
                    for dmin_s, ds_s in sub_data:
                        shit = jnp.any(dmin_s < st[2])

                        def do_sub(st, ds_s=ds_s):
                            for d, off in ds_s:
                                hit = jnp.any(d < st[2])
                                st = lax.cond(
                                    hit,
                                    lambda a, d=d, off=off: merge(a, d, off),
                                    lambda a: a, st)
                            return st

                        st = lax.cond(shit, do_sub, lambda a: a, st)
                    return st

                return lax.cond(ghit, do_group, lambda a: a, st)

            bd, bi, _ = lax.fori_loop(0, NGROUPS, group_step, st)

            nidx_v[...] = bi
            pltpu.async_copy(fun_hbm.at[nidx_v], rows_v, sem).wait()
            acc = [jnp.zeros((L,), jnp.float32) for _ in range(FV)]
            for r in range(K):
                for j in range(FV):
                    acc[j] = acc[j] + rows_v[r, pl.ds(j * L, L)]
            sn = g * L + lane
            scale = jnp.float32(1.0 / K)
            for j in range(FV):
                feats_v[sn, pl.ds(j * L, L)] = acc[j] * scale
            posv = jnp.where(
                lane_iota == 0, _splat_lane(sxv, lane_iota, lane),
                jnp.where(lane_iota == 1, _splat_lane(syv, lane_iota, lane),
                          jnp.where(lane_iota == 2,
                                    _splat_lane(szv, lane_iota, lane),
                                    zero16)))
            feats_v[sn, pl.ds(FUN, L)] = posv
            for j in range(FV + 1, DPAD // L):
                feats_v[sn, pl.ds(j * L, L)] = zero16
            return carry

        lax.fori_loop(0, L, sn_step, 0)

    pltpu.sync_copy(feats_v, feats_hbm.at[pl.ds(base, SN_PER_W)])


def _mm_body(f_ref, w_ref, b_ref, o_ref):
    o_ref[...] = (
        jax.lax.dot_general(
            f_ref[...], w_ref[...], (((1,), (0,)), ((), ())),
            preferred_element_type=jnp.float32,
            precision=jax.lax.Precision.HIGHEST)
        + b_ref[...]
    )


def kernel(pos, fun, supernode_idx, W, b):
    pos32 = pos.astype(jnp.float32)
    xs, ys, zs = pos32[:, 0], pos32[:, 1], pos32[:, 2]
    sidx = supernode_idx.astype(jnp.int32)
    fun32 = jnp.pad(fun.astype(jnp.float32), ((0, 0), (0, FPAD - FUN)))

    mesh = plsc.VectorSubcoreMesh(
        core_axis_name="c", subcore_axis_name="s",
        num_cores=NC, num_subcores=NS)
    sc = pl.kernel(
        _sc_body,
        out_type=[
            jax.ShapeDtypeStruct((S, DPAD), jnp.float32),
        ],
        mesh=mesh,
        scratch_types=[
            pltpu.VMEM((N,), jnp.float32),
            pltpu.VMEM((N,), jnp.float32),
            pltpu.VMEM((N,), jnp.float32),
            pltpu.VMEM((N,), jnp.float32),
            pltpu.VMEM((SN_PER_W,), jnp.int32),
            pltpu.VMEM((SPACE, SN_PER_W), jnp.float32),
            pltpu.VMEM((SPACE, SN_PER_W), jnp.float32),
            pltpu.VMEM((SN_PER_W, DPAD), jnp.float32),
            pltpu.VMEM((K,), jnp.int32),
            pltpu.VMEM((K, FPAD), jnp.float32),
            pltpu.SemaphoreType.DMA,
        ],
        compiler_params=pltpu.CompilerParams(needs_layout_passes=False),
    )
    (feats,) = sc(xs, ys, zs, sidx, fun32)

    W32 = W.astype(jnp.float32)
    Wp = jnp.concatenate(
        [W32[:, SPACE:], W32[:, :SPACE],
         jnp.zeros((CH, DPAD - SPACE - FUN), jnp.float32)], axis=1).T

    latent = pl.pallas_call(
        _mm_body,
        out_shape=jax.ShapeDtypeStruct((S, CH), jnp.float32),
    )(feats, Wp, b.astype(jnp.float32).reshape(1, CH))
    return latent


# speedup vs baseline: 4.3274x; 1.4128x over previous
import functools

import jax
import jax.numpy as jnp
from jax import lax
from jax.experimental import pallas as pl
from jax.experimental.pallas import tpu as pltpu
from jax.experimental.pallas import tpu_sc as plsc

N = 20000
S = 2048
SPACE = 3
FUN = 64
CH = 256
K = 16

L = 16
NC = 2
NS = 16
NW = NC * NS
SN_PER_W = S // NW
CHUNKS = N // L
WARM = 50
GCH = 25
NSUB = 5
SUBCH = GCH // NSUB
NGROUPS = (CHUNKS - WARM) // GCH
GROUPS = SN_PER_W // L
FV = FUN // L
FPAD = 128
DPAD = 128


def _splat_lane(vec, lane_iota, lane):
    m = lane_iota == lane
    val = jnp.sum(jnp.where(m, vec, jnp.float32(0.0)))
    return jnp.full((L,), val, jnp.float32)


def _round_bf16(v):
    u = lax.bitcast_convert_type(v, jnp.uint32)
    lsb = lax.shift_right_logical(u, jnp.uint32(16)) & jnp.uint32(1)
    r = u + jnp.uint32(0x7FFF) + lsb
    return lax.bitcast_convert_type(r & jnp.uint32(0xFFFF0000), jnp.float32)


def _sc_body(x_hbm, y_hbm, z_hbm, sidx_hbm, fun_hbm, feats_hbm,
             x_v, y_v, z_v, pn_v, sidx_v, sp_v, sp2_v, feats_v,
             nidx_v, rows_v, sem):
    cid = lax.axis_index("c")
    sid = lax.axis_index("s")
    wid = sid * NC + cid
    base = wid * SN_PER_W

    pltpu.sync_copy(x_hbm, x_v)
    pltpu.sync_copy(y_hbm, y_v)
    pltpu.sync_copy(z_hbm, z_v)
    pltpu.sync_copy(sidx_hbm.at[pl.ds(base, SN_PER_W)], sidx_v)

    for g in range(GROUPS):
        idx16 = sidx_v[pl.ds(g * L, L)]
        sx = plsc.load_gather(x_v, [idx16])
        sy = plsc.load_gather(y_v, [idx16])
        sz = plsc.load_gather(z_v, [idx16])
        sp_v[0, pl.ds(g * L, L)] = sx
        sp_v[1, pl.ds(g * L, L)] = sy
        sp_v[2, pl.ds(g * L, L)] = sz
        sxr = _round_bf16(sx)
        syr = _round_bf16(sy)
        szr = _round_bf16(sz)
        sp2_v[0, pl.ds(g * L, L)] = sxr + sxr
        sp2_v[1, pl.ds(g * L, L)] = syr + syr
        sp2_v[2, pl.ds(g * L, L)] = szr + szr

    def pn_step(c, carry):
        off = c * L
        xv = x_v[pl.ds(off, L)]
        yv = y_v[pl.ds(off, L)]
        zv = z_v[pl.ds(off, L)]
        pn_v[pl.ds(off, L)] = xv * xv + yv * yv + zv * zv
        x_v[pl.ds(off, L)] = _round_bf16(xv)
        y_v[pl.ds(off, L)] = _round_bf16(yv)
        z_v[pl.ds(off, L)] = _round_bf16(zv)
        return carry
    lax.fori_loop(0, CHUNKS, pn_step, 0)

    lane_iota = lax.iota(jnp.int32, L)
    inf16 = jnp.full((L,), jnp.inf, jnp.float32)
    zero16 = jnp.zeros((L,), jnp.float32)

    for g in range(GROUPS):
        sxv = sp_v[0, pl.ds(g * L, L)]
        syv = sp_v[1, pl.ds(g * L, L)]
        szv = sp_v[2, pl.ds(g * L, L)]
        sx2v = sp2_v[0, pl.ds(g * L, L)]
        sy2v = sp2_v[1, pl.ds(g * L, L)]
        sz2v = sp2_v[2, pl.ds(g * L, L)]

        def sn_step(lane, carry, sxv=sxv, syv=syv, szv=szv,
                    sx2v=sx2v, sy2v=sy2v, sz2v=sz2v, g=g):
            sx2 = _splat_lane(sx2v, lane_iota, lane)
            sy2 = _splat_lane(sy2v, lane_iota, lane)
            sz2 = _splat_lane(sz2v, lane_iota, lane)

            def dist(off):
                xv = x_v[pl.ds(off, L)]
                yv = y_v[pl.ds(off, L)]
                zv = z_v[pl.ds(off, L)]
                pnv = pn_v[pl.ds(off, L)]
                t = sx2 * xv + sy2 * yv + sz2 * zv
                return pnv - t

            def merge_vec(st, d, idxv):
                bd, bi, _ = st
                nd, ni = plsc.sort_key_val(d, idxv)
                ndr = lax.rev(nd, (0,))
                nir = lax.rev(ni, (0,))
                take = bd <= ndr
                lo_d = jnp.where(take, bd, ndr)
                lo_i = jnp.where(take, bi, nir)
                bd2, bi2 = plsc.sort_key_val(lo_d, lo_i)
                thr2 = plsc.cummax(lax.rev(plsc.cummax(lo_d), (0,)))
                return bd2, bi2, thr2

            def warm_step(c, st):
                off = c * L
                d = dist(off)
                hit = jnp.any(d < st[2])
                return lax.cond(
                    hit, lambda a: merge_vec(a, d, lane_iota + off),
                    lambda a: a, st)

            st = lax.fori_loop(
                0, WARM, warm_step,
                (inf16, jnp.zeros((L,), jnp.int32), inf16))

            def group_step(gi, st):
                off0 = (WARM + gi * GCH) * L
                b1 = inf16
                b2 = inf16
                i1 = jnp.zeros((L,), jnp.int32)
                offs = []
                for k in range(GCH):
                    off = off0 + k * L
                    offs.append(off)
                    d = dist(off)
                    idxv = lane_iota + off
                    m1 = d < b1
                    b2 = jnp.minimum(b2, jnp.where(m1, b1, d))
                    b1 = jnp.where(m1, d, b1)
                    i1 = jnp.where(m1, idxv, i1)
                ghit = jnp.any(b1 < st[2])

                def do_group(st, b1=b1, b2=b2, i1=i1):
                    st = merge_vec(st, b1, i1)
                    hit2 = jnp.any(b2 < st[2])

                    def do_cascade(st):
                        for off in offs:
                            d = dist(off)
                            idxv = lane_iota + off
                            d = jnp.where(idxv == i1, jnp.inf, d)
                            hit = jnp.any(d < st[2])
                            st = lax.cond(
                                hit,
                                lambda a, d=d, idxv=idxv: merge_vec(a, d, idxv),
                                lambda a: a, st)
                        return st

                    return lax.cond(hit2, do_cascade, lambda a: a, st)

                return lax.cond(ghit, do_group, lambda a: a, st)

            bd, bi, _ = lax.fori_loop(0, NGROUPS, group_step, st)

            nidx_v[...] = bi
            pltpu.async_copy(fun_hbm.at[nidx_v], rows_v, sem).wait()
            acc = [jnp.zeros((L,), jnp.float32) for _ in range(FV)]
            for r in range(K):
                for j in range(FV):
                    acc[j] = acc[j] + rows_v[r, pl.ds(j * L, L)]
            sn = g * L + lane
            scale = jnp.float32(1.0 / K)
            for j in range(FV):
                feats_v[sn, pl.ds(j * L, L)] = acc[j] * scale
            posv = jnp.where(
                lane_iota == 0, _splat_lane(sxv, lane_iota, lane),
                jnp.where(lane_iota == 1, _splat_lane(syv, lane_iota, lane),
                          jnp.where(lane_iota == 2,
                                    _splat_lane(szv, lane_iota, lane),
                                    zero16)))
            feats_v[sn, pl.ds(FUN, L)] = posv
            for j in range(FV + 1, DPAD // L):
                feats_v[sn, pl.ds(j * L, L)] = zero16
            return carry

        lax.fori_loop(0, L, sn_step, 0)

    pltpu.sync_copy(feats_v, feats_hbm.at[pl.ds(base, SN_PER_W)])


def _mm_body(f_ref, w_ref, b_ref, o_ref):
    o_ref[...] = (
        jax.lax.dot_general(
            f_ref[...], w_ref[...], (((1,), (0,)), ((), ())),
            preferred_element_type=jnp.float32,
            precision=jax.lax.Precision.HIGHEST)
        + b_ref[...]
    )


def kernel(pos, fun, supernode_idx, W, b):
    pos32 = pos.astype(jnp.float32)
    xs, ys, zs = pos32[:, 0], pos32[:, 1], pos32[:, 2]
    sidx = supernode_idx.astype(jnp.int32)
    fun32 = jnp.pad(fun.astype(jnp.float32), ((0, 0), (0, FPAD - FUN)))

    mesh = plsc.VectorSubcoreMesh(
        core_axis_name="c", subcore_axis_name="s",
        num_cores=NC, num_subcores=NS)
    sc = pl.kernel(
        _sc_body,
        out_type=[
            jax.ShapeDtypeStruct((S, DPAD), jnp.float32),
        ],
        mesh=mesh,
        scratch_types=[
            pltpu.VMEM((N,), jnp.float32),
            pltpu.VMEM((N,), jnp.float32),
            pltpu.VMEM((N,), jnp.float32),
            pltpu.VMEM((N,), jnp.float32),
            pltpu.VMEM((SN_PER_W,), jnp.int32),
            pltpu.VMEM((SPACE, SN_PER_W), jnp.float32),
            pltpu.VMEM((SPACE, SN_PER_W), jnp.float32),
            pltpu.VMEM((SN_PER_W, DPAD), jnp.float32),
            pltpu.VMEM((K,), jnp.int32),
            pltpu.VMEM((K, FPAD), jnp.float32),
            pltpu.SemaphoreType.DMA,
        ],
        compiler_params=pltpu.CompilerParams(needs_layout_passes=False),
    )
    (feats,) = sc(xs, ys, zs, sidx, fun32)

    W32 = W.astype(jnp.float32)
    Wp = jnp.concatenate(
        [W32[:, SPACE:], W32[:, :SPACE],
         jnp.zeros((CH, DPAD - SPACE - FUN), jnp.float32)], axis=1).T

    latent = pl.pallas_call(
        _mm_body,
        out_shape=jax.ShapeDtypeStruct((S, CH), jnp.float32),
    )(feats, Wp, b.astype(jnp.float32).reshape(1, CH))
    return latent


# speedup vs baseline: 4.5210x; 1.0447x over previous
import functools

import jax
import jax.numpy as jnp
from jax import lax
from jax.experimental import pallas as pl
from jax.experimental.pallas import tpu as pltpu
from jax.experimental.pallas import tpu_sc as plsc

N = 20000
S = 2048
SPACE = 3
FUN = 64
CH = 256
K = 16

L = 16
NC = 2
NS = 16
NW = NC * NS
SN_PER_W = S // NW
CHUNKS = N // L
WARM = 50
WGCH = 5
GCH = 25
NGROUPS = (CHUNKS - WARM) // GCH
GROUPS = SN_PER_W // L
FV = FUN // L
FPAD = 128
DPAD = 128


def _splat_lane(vec, lane_iota, lane):
    m = lane_iota == lane
    val = jnp.sum(jnp.where(m, vec, jnp.float32(0.0)))
    return jnp.full((L,), val, jnp.float32)


def _round_bf16(v):
    u = lax.bitcast_convert_type(v, jnp.uint32)
    lsb = lax.shift_right_logical(u, jnp.uint32(16)) & jnp.uint32(1)
    r = u + jnp.uint32(0x7FFF) + lsb
    return lax.bitcast_convert_type(r & jnp.uint32(0xFFFF0000), jnp.float32)


def _sc_body(x_hbm, y_hbm, z_hbm, sidx_hbm, fun_hbm, feats_hbm,
             x_v, y_v, z_v, pn_v, sidx_v, sp_v, sp2_v, feats_v,
             nidx_v, rows_v, sem):
    cid = lax.axis_index("c")
    sid = lax.axis_index("s")
    wid = sid * NC + cid
    base = wid * SN_PER_W

    pltpu.sync_copy(x_hbm, x_v)
    pltpu.sync_copy(y_hbm, y_v)
    pltpu.sync_copy(z_hbm, z_v)
    pltpu.sync_copy(sidx_hbm.at[pl.ds(base, SN_PER_W)], sidx_v)

    for g in range(GROUPS):
        idx16 = sidx_v[pl.ds(g * L, L)]
        sx = plsc.load_gather(x_v, [idx16])
        sy = plsc.load_gather(y_v, [idx16])
        sz = plsc.load_gather(z_v, [idx16])
        sp_v[0, pl.ds(g * L, L)] = sx
        sp_v[1, pl.ds(g * L, L)] = sy
        sp_v[2, pl.ds(g * L, L)] = sz
        sxr = _round_bf16(sx)
        syr = _round_bf16(sy)
        szr = _round_bf16(sz)
        sp2_v[0, pl.ds(g * L, L)] = sxr + sxr
        sp2_v[1, pl.ds(g * L, L)] = syr + syr
        sp2_v[2, pl.ds(g * L, L)] = szr + szr

    def pn_step(c, carry):
        off = c * L
        xv = x_v[pl.ds(off, L)]
        yv = y_v[pl.ds(off, L)]
        zv = z_v[pl.ds(off, L)]
        pn_v[pl.ds(off, L)] = xv * xv + yv * yv + zv * zv
        x_v[pl.ds(off, L)] = _round_bf16(xv)
        y_v[pl.ds(off, L)] = _round_bf16(yv)
        z_v[pl.ds(off, L)] = _round_bf16(zv)
        return carry
    lax.fori_loop(0, CHUNKS, pn_step, 0)

    lane_iota = lax.iota(jnp.int32, L)
    inf16 = jnp.full((L,), jnp.inf, jnp.float32)
    zero16 = jnp.zeros((L,), jnp.float32)

    for g in range(GROUPS):
        sxv = sp_v[0, pl.ds(g * L, L)]
        syv = sp_v[1, pl.ds(g * L, L)]
        szv = sp_v[2, pl.ds(g * L, L)]
        sx2v = sp2_v[0, pl.ds(g * L, L)]
        sy2v = sp2_v[1, pl.ds(g * L, L)]
        sz2v = sp2_v[2, pl.ds(g * L, L)]

        def sn_step(lane, carry, sxv=sxv, syv=syv, szv=szv,
                    sx2v=sx2v, sy2v=sy2v, sz2v=sz2v, g=g):
            sx2 = _splat_lane(sx2v, lane_iota, lane)
            sy2 = _splat_lane(sy2v, lane_iota, lane)
            sz2 = _splat_lane(sz2v, lane_iota, lane)

            def dist(off):
                xv = x_v[pl.ds(off, L)]
                yv = y_v[pl.ds(off, L)]
                zv = z_v[pl.ds(off, L)]
                pnv = pn_v[pl.ds(off, L)]
                t = sx2 * xv + sy2 * yv + sz2 * zv
                return pnv - t

            def merge_vec(st, d, idxv):
                bd, bi, _ = st
                nd, ni = plsc.sort_key_val(d, idxv)
                ndr = lax.rev(nd, (0,))
                nir = lax.rev(ni, (0,))
                take = bd <= ndr
                lo_d = jnp.where(take, bd, ndr)
                lo_i = jnp.where(take, bi, nir)
                bd2, bi2 = plsc.sort_key_val(lo_d, lo_i)
                thr2 = plsc.cummax(lax.rev(plsc.cummax(lo_d), (0,)))
                return bd2, bi2, thr2

            def run_group(st, off0, gch):
                b1 = inf16
                b2 = inf16
                i1 = jnp.zeros((L,), jnp.int32)
                offs = []
                for k in range(gch):
                    off = off0 + k * L
                    offs.append(off)
                    d = dist(off)
                    idxv = lane_iota + off
                    m1 = d < b1
                    b2 = jnp.minimum(b2, jnp.where(m1, b1, d))
                    b1 = jnp.where(m1, d, b1)
                    i1 = jnp.where(m1, idxv, i1)
                ghit = jnp.any(b1 < st[2])

                def do_group(st, b1=b1, b2=b2, i1=i1):
                    st = merge_vec(st, b1, i1)
                    hit2 = jnp.any(b2 < st[2])

                    def do_cascade(st):
                        for off in offs:
                            d = dist(off)
                            idxv = lane_iota + off
                            d = jnp.where(idxv == i1, jnp.inf, d)
                            hit = jnp.any(d < st[2])
                            st = lax.cond(
                                hit,
                                lambda a, d=d, idxv=idxv: merge_vec(a, d, idxv),
                                lambda a: a, st)
                        return st

                    return lax.cond(hit2, do_cascade, lambda a: a, st)

                return lax.cond(ghit, do_group, lambda a: a, st)

            st0 = (inf16, jnp.zeros((L,), jnp.int32), inf16)
            st = lax.fori_loop(
                0, WARM // WGCH,
                lambda gi, st: run_group(st, gi * (WGCH * L), WGCH), st0)
            bd, bi, _ = lax.fori_loop(
                0, NGROUPS,
                lambda gi, st: run_group(st, (WARM + gi * GCH) * L, GCH), st)

            nidx_v[...] = bi
            pltpu.async_copy(fun_hbm.at[nidx_v], rows_v, sem).wait()
            acc = [jnp.zeros((L,), jnp.float32) for _ in range(FV)]
            for r in range(K):
                for j in range(FV):
                    acc[j] = acc[j] + rows_v[r, pl.ds(j * L, L)]
            sn = g * L + lane
            scale = jnp.float32(1.0 / K)
            for j in range(FV):
                feats_v[sn, pl.ds(j * L, L)] = acc[j] * scale
            posv = jnp.where(
                lane_iota == 0, _splat_lane(sxv, lane_iota, lane),
                jnp.where(lane_iota == 1, _splat_lane(syv, lane_iota, lane),
                          jnp.where(lane_iota == 2,
                                    _splat_lane(szv, lane_iota, lane),
                                    zero16)))
            feats_v[sn, pl.ds(FUN, L)] = posv
            for j in range(FV + 1, DPAD // L):
                feats_v[sn, pl.ds(j * L, L)] = zero16
            return carry

        lax.fori_loop(0, L, sn_step, 0)

    pltpu.sync_copy(feats_v, feats_hbm.at[pl.ds(base, SN_PER_W)])


def _mm_body(f_ref, w_ref, b_ref, o_ref):
    o_ref[...] = (
        jax.lax.dot_general(
            f_ref[...], w_ref[...], (((1,), (0,)), ((), ())),
            preferred_element_type=jnp.float32,
            precision=jax.lax.Precision.HIGHEST)
        + b_ref[...]
    )


def kernel(pos, fun, supernode_idx, W, b):
    pos32 = pos.astype(jnp.float32)
    xs, ys, zs = pos32[:, 0], pos32[:, 1], pos32[:, 2]
    sidx = supernode_idx.astype(jnp.int32)
    fun32 = jnp.pad(fun.astype(jnp.float32), ((0, 0), (0, FPAD - FUN)))

    mesh = plsc.VectorSubcoreMesh(
        core_axis_name="c", subcore_axis_name="s",
        num_cores=NC, num_subcores=NS)
    sc = pl.kernel(
        _sc_body,
        out_type=[
            jax.ShapeDtypeStruct((S, DPAD), jnp.float32),
        ],
        mesh=mesh,
        scratch_types=[
            pltpu.VMEM((N,), jnp.float32),
            pltpu.VMEM((N,), jnp.float32),
            pltpu.VMEM((N,), jnp.float32),
            pltpu.VMEM((N,), jnp.float32),
            pltpu.VMEM((SN_PER_W,), jnp.int32),
            pltpu.VMEM((SPACE, SN_PER_W), jnp.float32),
            pltpu.VMEM((SPACE, SN_PER_W), jnp.float32),
            pltpu.VMEM((SN_PER_W, DPAD), jnp.float32),
            pltpu.VMEM((K,), jnp.int32),
            pltpu.VMEM((K, FPAD), jnp.float32),
            pltpu.SemaphoreType.DMA,
        ],
        compiler_params=pltpu.CompilerParams(needs_layout_passes=False),
    )
    (feats,) = sc(xs, ys, zs, sidx, fun32)

    W32 = W.astype(jnp.float32)
    Wp = jnp.concatenate(
        [W32[:, SPACE:], W32[:, :SPACE],
         jnp.zeros((CH, DPAD - SPACE - FUN), jnp.float32)], axis=1).T

    latent = pl.pallas_call(
        _mm_body,
        out_shape=jax.ShapeDtypeStruct((S, CH), jnp.float32),
    )(feats, Wp, b.astype(jnp.float32).reshape(1, CH))
    return latent


# speedup vs baseline: 4.6975x; 1.0390x over previous
import functools

import jax
import jax.numpy as jnp
from jax import lax
from jax.experimental import pallas as pl
from jax.experimental.pallas import tpu as pltpu
from jax.experimental.pallas import tpu_sc as plsc

N = 20000
S = 2048
SPACE = 3
FUN = 64
CH = 256
K = 16

L = 16
NC = 2
NS = 16
NW = NC * NS
SN_PER_W = S // NW
CHUNKS = N // L
WARM = 100
WGCH = 10
GCH = 25
NGROUPS = (CHUNKS - WARM) // GCH
GROUPS = SN_PER_W // L
FV = FUN // L
FPAD = 128
DPAD = 128


def _splat_lane(vec, lane_iota, lane):
    m = lane_iota == lane
    val = jnp.sum(jnp.where(m, vec, jnp.float32(0.0)))
    return jnp.full((L,), val, jnp.float32)


def _round_bf16(v):
    u = lax.bitcast_convert_type(v, jnp.uint32)
    lsb = lax.shift_right_logical(u, jnp.uint32(16)) & jnp.uint32(1)
    r = u + jnp.uint32(0x7FFF) + lsb
    return lax.bitcast_convert_type(r & jnp.uint32(0xFFFF0000), jnp.float32)


def _sc_body(x_hbm, y_hbm, z_hbm, sidx_hbm, fun_hbm, feats_hbm,
             x_v, y_v, z_v, pn_v, sidx_v, sp_v, sp2_v, feats_v,
             nidx_v, rows_v, sem):
    cid = lax.axis_index("c")
    sid = lax.axis_index("s")
    wid = sid * NC + cid
    base = wid * SN_PER_W

    pltpu.sync_copy(x_hbm, x_v)
    pltpu.sync_copy(y_hbm, y_v)
    pltpu.sync_copy(z_hbm, z_v)
    pltpu.sync_copy(sidx_hbm.at[pl.ds(base, SN_PER_W)], sidx_v)

    for g in range(GROUPS):
        idx16 = sidx_v[pl.ds(g * L, L)]
        sx = plsc.load_gather(x_v, [idx16])
        sy = plsc.load_gather(y_v, [idx16])
        sz = plsc.load_gather(z_v, [idx16])
        sp_v[0, pl.ds(g * L, L)] = sx
        sp_v[1, pl.ds(g * L, L)] = sy
        sp_v[2, pl.ds(g * L, L)] = sz
        sxr = _round_bf16(sx)
        syr = _round_bf16(sy)
        szr = _round_bf16(sz)
        sp2_v[0, pl.ds(g * L, L)] = sxr + sxr
        sp2_v[1, pl.ds(g * L, L)] = syr + syr
        sp2_v[2, pl.ds(g * L, L)] = szr + szr

    def pn_step(c, carry):
        off = c * L
        xv = x_v[pl.ds(off, L)]
        yv = y_v[pl.ds(off, L)]
        zv = z_v[pl.ds(off, L)]
        pn_v[pl.ds(off, L)] = xv * xv + yv * yv + zv * zv
        x_v[pl.ds(off, L)] = _round_bf16(xv)
        y_v[pl.ds(off, L)] = _round_bf16(yv)
        z_v[pl.ds(off, L)] = _round_bf16(zv)
        return carry
    lax.fori_loop(0, CHUNKS, pn_step, 0)

    lane_iota = lax.iota(jnp.int32, L)
    inf16 = jnp.full((L,), jnp.inf, jnp.float32)
    zero16 = jnp.zeros((L,), jnp.float32)

    for g in range(GROUPS):
        sxv = sp_v[0, pl.ds(g * L, L)]
        syv = sp_v[1, pl.ds(g * L, L)]
        szv = sp_v[2, pl.ds(g * L, L)]
        sx2v = sp2_v[0, pl.ds(g * L, L)]
        sy2v = sp2_v[1, pl.ds(g * L, L)]
        sz2v = sp2_v[2, pl.ds(g * L, L)]

        def sn_step(lane, carry, sxv=sxv, syv=syv, szv=szv,
                    sx2v=sx2v, sy2v=sy2v, sz2v=sz2v, g=g):
            sx2 = _splat_lane(sx2v, lane_iota, lane)
            sy2 = _splat_lane(sy2v, lane_iota, lane)
            sz2 = _splat_lane(sz2v, lane_iota, lane)

            def dist(off):
                xv = x_v[pl.ds(off, L)]
                yv = y_v[pl.ds(off, L)]
                zv = z_v[pl.ds(off, L)]
                pnv = pn_v[pl.ds(off, L)]
                t = sx2 * xv + sy2 * yv + sz2 * zv
                return pnv - t

            def merge_vec(st, d, idxv):
                bd, bi, _ = st
                nd, ni = plsc.sort_key_val(d, idxv)
                ndr = lax.rev(nd, (0,))
                nir = lax.rev(ni, (0,))
                take = bd <= ndr
                lo_d = jnp.where(take, bd, ndr)
                lo_i = jnp.where(take, bi, nir)
                bd2, bi2 = plsc.sort_key_val(lo_d, lo_i)
                thr2 = plsc.cummax(lax.rev(plsc.cummax(lo_d), (0,)))
                return bd2, bi2, thr2

            def run_group(st, off0, gch):
                b1 = inf16
                b2 = inf16
                i1 = jnp.zeros((L,), jnp.int32)
                offs = []
                for k in range(gch):
                    off = off0 + k * L
                    offs.append(off)
                    d = dist(off)
                    idxv = lane_iota + off
                    m1 = d < b1
                    b2 = jnp.minimum(b2, jnp.where(m1, b1, d))
                    b1 = jnp.where(m1, d, b1)
                    i1 = jnp.where(m1, idxv, i1)
                ghit = jnp.any(b1 < st[2])

                def do_group(st, b1=b1, b2=b2, i1=i1):
                    st = merge_vec(st, b1, i1)
                    hit2 = jnp.any(b2 < st[2])

                    def do_cascade(st):
                        for off in offs:
                            d = dist(off)
                            idxv = lane_iota + off
                            d = jnp.where(idxv == i1, jnp.inf, d)
                            hit = jnp.any(d < st[2])
                            st = lax.cond(
                                hit,
                                lambda a, d=d, idxv=idxv: merge_vec(a, d, idxv),
                                lambda a: a, st)
                        return st

                    return lax.cond(hit2, do_cascade, lambda a: a, st)

                return lax.cond(ghit, do_group, lambda a: a, st)

            st0 = (inf16, jnp.zeros((L,), jnp.int32), inf16)
            st = lax.fori_loop(
                0, WARM // WGCH,
                lambda gi, st: run_group(st, gi * (WGCH * L), WGCH), st0)
            bd, bi, _ = lax.fori_loop(
                0, NGROUPS,
                lambda gi, st: run_group(st, (WARM + gi * GCH) * L, GCH), st)

            nidx_v[...] = bi
            pltpu.async_copy(fun_hbm.at[nidx_v], rows_v, sem).wait()
            acc = [jnp.zeros((L,), jnp.float32) for _ in range(FV)]
            for r in range(K):
                for j in range(FV):
                    acc[j] = acc[j] + rows_v[r, pl.ds(j * L, L)]
            sn = g * L + lane
            scale = jnp.float32(1.0 / K)
            for j in range(FV):
                feats_v[sn, pl.ds(j * L, L)] = acc[j] * scale
            posv = jnp.where(
                lane_iota == 0, _splat_lane(sxv, lane_iota, lane),
                jnp.where(lane_iota == 1, _splat_lane(syv, lane_iota, lane),
                          jnp.where(lane_iota == 2,
                                    _splat_lane(szv, lane_iota, lane),
                                    zero16)))
            feats_v[sn, pl.ds(FUN, L)] = posv
            for j in range(FV + 1, DPAD // L):
                feats_v[sn, pl.ds(j * L, L)] = zero16
            return carry

        lax.fori_loop(0, L, sn_step, 0)

    pltpu.sync_copy(feats_v, feats_hbm.at[pl.ds(base, SN_PER_W)])


def _mm_body(f_ref, w_ref, b_ref, o_ref):
    o_ref[...] = (
        jax.lax.dot_general(
            f_ref[...], w_ref[...], (((1,), (0,)), ((), ())),
            preferred_element_type=jnp.float32,
            precision=jax.lax.Precision.HIGHEST)
        + b_ref[...]
    )


def kernel(pos, fun, supernode_idx, W, b):
    pos32 = pos.astype(jnp.float32)
    xs, ys, zs = pos32[:, 0], pos32[:, 1], pos32[:, 2]
    sidx = supernode_idx.astype(jnp.int32)
    fun32 = jnp.pad(fun.astype(jnp.float32), ((0, 0), (0, FPAD - FUN)))

    mesh = plsc.VectorSubcoreMesh(
        core_axis_name="c", subcore_axis_name="s",
        num_cores=NC, num_subcores=NS)
    sc = pl.kernel(
        _sc_body,
        out_type=[
            jax.ShapeDtypeStruct((S, DPAD), jnp.float32),
        ],
        mesh=mesh,
        scratch_types=[
            pltpu.VMEM((N,), jnp.float32),
            pltpu.VMEM((N,), jnp.float32),
            pltpu.VMEM((N,), jnp.float32),
            pltpu.VMEM((N,), jnp.float32),
            pltpu.VMEM((SN_PER_W,), jnp.int32),
            pltpu.VMEM((SPACE, SN_PER_W), jnp.float32),
            pltpu.VMEM((SPACE, SN_PER_W), jnp.float32),
            pltpu.VMEM((SN_PER_W, DPAD), jnp.float32),
            pltpu.VMEM((K,), jnp.int32),
            pltpu.VMEM((K, FPAD), jnp.float32),
            pltpu.SemaphoreType.DMA,
        ],
        compiler_params=pltpu.CompilerParams(needs_layout_passes=False),
    )
    (feats,) = sc(xs, ys, zs, sidx, fun32)

    W32 = W.astype(jnp.float32)
    Wp = jnp.concatenate(
        [W32[:, SPACE:], W32[:, :SPACE],
         jnp.zeros((CH, DPAD - SPACE - FUN), jnp.float32)], axis=1).T

    latent = pl.pallas_call(
        _mm_body,
        out_shape=jax.ShapeDtypeStruct((S, CH), jnp.float32),
    )(feats, Wp, b.astype(jnp.float32).reshape(1, CH))
    return latent
